# Initial kernel scaffold; baseline (speedup 1.0000x reference)
import jax
import jax.numpy as jnp
from jax.experimental import pallas as pl


def kernel(x, embed, W, b):
    raise NotImplementedError("write your pallas kernel here")


# speedup vs baseline: 6.7131x; 6.7131x over previous
import functools

import jax
import jax.numpy as jnp
from jax import lax
from jax.experimental import pallas as pl
from jax.experimental.pallas import tpu as pltpu
from jax.experimental.pallas import tpu_sc as plsc

_NC = 2
_NS = 16
_NW = _NC * _NS
_LANES = 16


def _matvec_body(e_ref, w_ref, b_ref, t_ref):
    t_ref[...] = jnp.sum(e_ref[...] * w_ref[...], axis=1, keepdims=True) + b_ref[0]


def _precompute_table(embed, W, b):
    V, D = embed.shape
    br = next(c for c in (2000, 1000, 500, 250, 125, 8, 1) if V % c == 0)
    t = pl.pallas_call(
        _matvec_body,
        grid=(V // br,),
        in_specs=[
            pl.BlockSpec((br, D), lambda i: (i, 0)),
            pl.BlockSpec((1, D), lambda i: (0, 0)),
            pl.BlockSpec(memory_space=pltpu.SMEM),
        ],
        out_specs=pl.BlockSpec((br, 1), lambda i: (i, 0)),
        out_shape=jax.ShapeDtypeStruct((V, 1), jnp.float32),
    )(embed, W, b)
    return t.reshape(V)


@functools.lru_cache(maxsize=None)
def _make_pool_kernel(B, L):
    n_rows = B // _NW
    n_idx = n_rows * L
    assert n_idx % 128 == 0 and n_rows % _LANES == 0
    n_chunks = n_idx // 128
    mesh = plsc.VectorSubcoreMesh(core_axis_name="c", subcore_axis_name="s")

    @functools.partial(
        pl.kernel,
        out_type=jax.ShapeDtypeStruct((B,), jnp.float32),
        mesh=mesh,
        scratch_types=[
            pltpu.VMEM((n_idx,), jnp.int32),
            pltpu.VMEM((n_idx,), jnp.float32),
            pltpu.VMEM((n_rows,), jnp.float32),
            pltpu.SemaphoreType.DMA,
        ],
    )
    def pool(x_hbm, t_hbm, out_hbm, idx_v, vals_v, out_v, sem):
        wid = lax.axis_index("s") * _NC + lax.axis_index("c")
        pltpu.sync_copy(x_hbm.at[pl.ds(wid * n_idx, n_idx)], idx_v)
        copies = [
            pltpu.async_copy(
                t_hbm.at[idx_v.at[pl.ds(c * 128, 128)]],
                vals_v.at[pl.ds(c * 128, 128)],
                sem,
            )
            for c in range(n_chunks)
        ]
        for cp in copies:
            cp.wait()
        inv_l = 1.0 / L
        for j in range(n_rows // _LANES):
            col = j * _LANES

            def body(l, acc):
                return acc + vals_v[pl.ds(l * n_rows + col, _LANES)]

            acc = lax.fori_loop(0, L, body, jnp.zeros((_LANES,), jnp.float32))
            out_v[pl.ds(col, _LANES)] = 1.0 / (1.0 + jnp.exp(acc * -inv_l))
        pltpu.sync_copy(out_v, out_hbm.at[pl.ds(wid * n_rows, n_rows)])

    return pool


def kernel(x, embed, W, b):
    B, L = x.shape
    t = _precompute_table(embed, W.astype(jnp.float32), b.astype(jnp.float32))
    n_rows = B // _NW
    xf = (
        x.astype(jnp.int32)
        .reshape(_NW, n_rows, L)
        .transpose(0, 2, 1)
        .reshape(B * L)
    )
    out = _make_pool_kernel(B, L)(xf, t)
    return out.reshape(B, 1)
